# Initial kernel scaffold; baseline (speedup 1.0000x reference)
import functools

import jax
import jax.numpy as jnp
import numpy as np
from jax import lax
from jax.experimental import pallas as pl
from jax.experimental.pallas import tpu as pltpu
from jax.experimental.pallas import tpu_sc as plsc

_NC, _NS = 2, 16
_NW = _NC * _NS
_CH = 128
_BS = 2048
_CLIP = 1.0 + 1e-7
_LOG2 = float(np.log(2.0))


def _sc_gather(table, iu, iv):
    n, d = table.shape
    b = iu.shape[0]
    bpw = b // _NW
    nch = bpw // _CH
    idx_u = iu.reshape(_NW, nch, _CH)
    idx_v = iv.reshape(_NW, nch, _CH)
    mesh = plsc.VectorSubcoreMesh(core_axis_name="c", subcore_axis_name="s")

    @functools.partial(
        pl.kernel,
        mesh=mesh,
        out_type=(
            jax.ShapeDtypeStruct((b, d), jnp.float32),
            jax.ShapeDtypeStruct((b, d), jnp.float32),
        ),
        scratch_types=[
            pltpu.VMEM((nch, _CH), jnp.int32),
            pltpu.VMEM((nch, _CH), jnp.int32),
            pltpu.VMEM((bpw, d), jnp.float32),
            pltpu.VMEM((bpw, d), jnp.float32),
            pltpu.SemaphoreType.DMA,
            pltpu.SemaphoreType.DMA,
        ],
    )
    def k(table_h, iu_h, iv_h, us_h, vs_h, iu_m, iv_m, us_m, vs_m, semu, semv):
        wid = lax.axis_index("s") * _NC + lax.axis_index("c")
        base = wid * bpw
        pltpu.sync_copy(iu_h.at[wid], iu_m)
        pltpu.sync_copy(iv_h.at[wid], iv_m)
        cps = []
        for c in range(nch):
            cps.append(pltpu.async_copy(
                table_h.at[iu_m.at[c]], us_m.at[pl.ds(c * _CH, _CH)], semu))
            cps.append(pltpu.async_copy(
                table_h.at[iv_m.at[c]], vs_m.at[pl.ds(c * _CH, _CH)], semv))
        for cp in cps:
            cp.wait()
        pltpu.sync_copy(us_m, us_h.at[pl.ds(base, bpw)])
        pltpu.sync_copy(vs_m, vs_h.at[pl.ds(base, bpw)])

    return k(table, idx_u, idx_v)


def _tc_body(nd, p_ref, ut_ref, vt_ref, si_ref, lab_ref, out_ref):
    c0 = p_ref[0]
    sq = p_ref[1]
    gamma = p_ref[2]
    inv_nm1 = p_ref[3]
    nd_m1 = float(nd - 1)
    u0 = ut_ref[0:1, :]
    v0 = vt_ref[0:1, :]
    usp = ut_ref[1:nd + 1, :]
    vsp = vt_ref[1:nd + 1, :]
    si = si_ref[...]
    alpha = u0 * v0 - jnp.sum(usp * vsp, axis=0, keepdims=True)
    x = jnp.maximum(alpha, _CLIP)
    dist = jnp.log(x + jnp.sqrt((x - 1.0) * (x + 1.0)))
    lab = lab_ref[...].astype(jnp.float32)
    z = sq * dist - gamma
    sz = jnp.where(lab == 1.0, z, -z)
    loss = jnp.maximum(sz, 0.0) + jnp.log(1.0 + jnp.exp(-jnp.abs(sz)))

    def lik(z0, zsp):
        su_w = jnp.sum(zsp * zsp * si, axis=0, keepdims=True)
        su2 = jnp.sum(zsp * zsp, axis=0, keepdims=True)
        ac = jnp.maximum(z0, _CLIP)
        den2 = ac * ac - 1.0
        denom = jnp.sqrt(den2)
        coef = jnp.log(ac + denom) / denom
        lk = c0 + 0.5 * coef * coef * su_w
        vn = jnp.sqrt(jnp.maximum(coef * coef * su2, 1e-12))
        vn = jnp.where(vn <= 1e-6, 1e-6, vn)
        return lk + nd_m1 * (jnp.log(1.0 - jnp.exp(-2.0 * vn)) + vn - _LOG2
                             - jnp.log(vn))

    out_ref[...] = loss + (lik(u0, usp) + lik(v0, vsp)) * inv_nm1


def _tc_math(ut, vt, siginv2, lab2, params, nd, b):
    body = functools.partial(_tc_body, nd)
    return pl.pallas_call(
        body,
        grid=(b // _BS,),
        in_specs=[
            pl.BlockSpec(memory_space=pltpu.SMEM),
            pl.BlockSpec((nd + 1, _BS), lambda i: (0, i)),
            pl.BlockSpec((nd + 1, _BS), lambda i: (0, i)),
            pl.BlockSpec((nd, 1), lambda i: (0, 0)),
            pl.BlockSpec((1, _BS), lambda i: (0, i)),
        ],
        out_specs=pl.BlockSpec((1, _BS), lambda i: (0, i)),
        out_shape=jax.ShapeDtypeStruct((1, b), jnp.float32),
    )(params, ut, vt, siginv2, lab2)


def kernel(table, sigma, kappa, gamma, pairs, labels):
    n, d = table.shape
    nd = d - 1
    b = pairs.shape[0]
    iu = pairs[:, 0]
    iv = pairs[:, 1]
    us, vs = _sc_gather(table, iu, iv)
    c0 = (nd / 2.0 * jnp.log(jnp.asarray(2.0 * np.pi, jnp.float32))
          + 0.5 * jnp.sum(jnp.log(sigma))
          + nd / 2.0 * jnp.log(jnp.abs(kappa)))
    sq = jnp.sqrt(jnp.abs(kappa))
    inv_nm1 = jnp.asarray(1.0 / (n - 1), jnp.float32)
    params = jnp.stack([c0, sq, gamma.astype(jnp.float32), inv_nm1])
    siginv2 = (1.0 / sigma).reshape(nd, 1)
    lab2 = labels.reshape(1, b)
    loss2 = _tc_math(us.T, vs.T, siginv2, lab2, params, nd, b)
    return loss2.reshape(b)


# speedup vs baseline: 1.0429x; 1.0429x over previous
import functools

import jax
import jax.numpy as jnp
import numpy as np
from jax import lax
from jax.experimental import pallas as pl
from jax.experimental.pallas import tpu as pltpu
from jax.experimental.pallas import tpu_sc as plsc

_NC, _NS = 2, 16
_NW = _NC * _NS
_CH = 128
_BS = 2048
_CLIP = 1.0 + 1e-7
_LOG2 = float(np.log(2.0))


def _sc_gather(table_t, iu, iv):
    d, n = table_t.shape
    b = iu.shape[0]
    bpw = b // _NW
    nch = bpw // _CH
    idx_u = iu.reshape(_NW, nch, _CH)
    idx_v = iv.reshape(_NW, nch, _CH)
    mesh = plsc.VectorSubcoreMesh(core_axis_name="c", subcore_axis_name="s")

    @functools.partial(
        pl.kernel,
        mesh=mesh,
        out_type=(
            jax.ShapeDtypeStruct((d, b), jnp.float32),
            jax.ShapeDtypeStruct((d, b), jnp.float32),
        ),
        scratch_types=[
            pltpu.VMEM((nch, _CH), jnp.int32),
            pltpu.VMEM((nch, _CH), jnp.int32),
            pltpu.VMEM((d, bpw), jnp.float32),
            pltpu.VMEM((d, bpw), jnp.float32),
            pltpu.SemaphoreType.DMA,
            pltpu.SemaphoreType.DMA,
        ],
    )
    def k(table_h, iu_h, iv_h, ut_h, vt_h, iu_m, iv_m, us_m, vs_m, semu, semv):
        wid = lax.axis_index("s") * _NC + lax.axis_index("c")
        base = wid * bpw
        pltpu.sync_copy(iu_h.at[wid], iu_m)
        pltpu.sync_copy(iv_h.at[wid], iv_m)
        cps = []
        for c in range(nch):
            for r in range(d):
                cps.append(pltpu.async_copy(
                    table_h.at[r].at[iu_m.at[c]],
                    us_m.at[r, pl.ds(c * _CH, _CH)], semu))
                cps.append(pltpu.async_copy(
                    table_h.at[r].at[iv_m.at[c]],
                    vs_m.at[r, pl.ds(c * _CH, _CH)], semv))
        for cp in cps:
            cp.wait()
        pltpu.sync_copy(us_m, ut_h.at[:, pl.ds(base, bpw)])
        pltpu.sync_copy(vs_m, vt_h.at[:, pl.ds(base, bpw)])

    return k(table_t, idx_u, idx_v)


def _tc_body(nd, p_ref, ut_ref, vt_ref, si_ref, lab_ref, out_ref):
    c0 = p_ref[0]
    sq = p_ref[1]
    gamma = p_ref[2]
    inv_nm1 = p_ref[3]
    nd_m1 = float(nd - 1)
    u0 = ut_ref[0:1, :]
    v0 = vt_ref[0:1, :]
    usp = ut_ref[1:nd + 1, :]
    vsp = vt_ref[1:nd + 1, :]
    si = si_ref[...]
    alpha = u0 * v0 - jnp.sum(usp * vsp, axis=0, keepdims=True)
    x = jnp.maximum(alpha, _CLIP)
    dist = jnp.log(x + jnp.sqrt((x - 1.0) * (x + 1.0)))
    lab = lab_ref[...].astype(jnp.float32)
    z = sq * dist - gamma
    sz = jnp.where(lab == 1.0, z, -z)
    loss = jnp.maximum(sz, 0.0) + jnp.log(1.0 + jnp.exp(-jnp.abs(sz)))

    def lik(z0, zsp):
        su_w = jnp.sum(zsp * zsp * si, axis=0, keepdims=True)
        su2 = jnp.sum(zsp * zsp, axis=0, keepdims=True)
        ac = jnp.maximum(z0, _CLIP)
        den2 = ac * ac - 1.0
        denom = jnp.sqrt(den2)
        coef = jnp.log(ac + denom) / denom
        lk = c0 + 0.5 * coef * coef * su_w
        vn = jnp.sqrt(jnp.maximum(coef * coef * su2, 1e-12))
        vn = jnp.where(vn <= 1e-6, 1e-6, vn)
        return lk + nd_m1 * (jnp.log(1.0 - jnp.exp(-2.0 * vn)) + vn - _LOG2
                             - jnp.log(vn))

    out_ref[...] = loss + (lik(u0, usp) + lik(v0, vsp)) * inv_nm1


def _tc_math(ut, vt, siginv2, lab2, params, nd, b):
    body = functools.partial(_tc_body, nd)
    return pl.pallas_call(
        body,
        grid=(b // _BS,),
        in_specs=[
            pl.BlockSpec(memory_space=pltpu.SMEM),
            pl.BlockSpec((nd + 1, _BS), lambda i: (0, i)),
            pl.BlockSpec((nd + 1, _BS), lambda i: (0, i)),
            pl.BlockSpec((nd, 1), lambda i: (0, 0)),
            pl.BlockSpec((1, _BS), lambda i: (0, i)),
        ],
        out_specs=pl.BlockSpec((1, _BS), lambda i: (0, i)),
        out_shape=jax.ShapeDtypeStruct((1, b), jnp.float32),
    )(params, ut, vt, siginv2, lab2)


def kernel(table, sigma, kappa, gamma, pairs, labels):
    n, d = table.shape
    nd = d - 1
    b = pairs.shape[0]
    iu = pairs[:, 0]
    iv = pairs[:, 1]
    ut = table[iu].T
    vt = table[iv].T
    c0 = (nd / 2.0 * jnp.log(jnp.asarray(2.0 * np.pi, jnp.float32))
          + 0.5 * jnp.sum(jnp.log(sigma))
          + nd / 2.0 * jnp.log(jnp.abs(kappa)))
    sq = jnp.sqrt(jnp.abs(kappa))
    inv_nm1 = jnp.asarray(1.0 / (n - 1), jnp.float32)
    params = jnp.stack([c0, sq, gamma.astype(jnp.float32), inv_nm1])
    siginv2 = (1.0 / sigma).reshape(nd, 1)
    lab2 = labels.reshape(1, b)
    loss2 = _tc_math(ut, vt, siginv2, lab2, params, nd, b)
    return loss2.reshape(b)
